# baseline (speedup 1.0000x reference)
import functools

import jax
import jax.numpy as jnp
from jax import lax
from jax.experimental import pallas as pl
from jax.experimental.pallas import tpu as pltpu
from jax.experimental.pallas import tpu_sc as plsc

_MAX_SIZE = 32
_L = 16


def _sc_body(emb_hbm, out_hbm, idx2, rows_v, bufs, gsem, csems, *, B, H, W, C):
    nc = 2
    ns = 16
    nw = nc * ns
    wb = B * H // nw
    cvecs = C // _L
    wid = lax.axis_index("c") * ns + lax.axis_index("s")
    b = wid // (nw // B)
    w8 = lax.rem(wid, nw // B)

    half = W // 2
    for r in range(2):
        for t in range(half // _L):
            base = r * half + t * _L
            iv = lax.iota(jnp.int32, _L) + (base - W // 2)
            idx2[r, pl.ds(t * _L, _L)] = (
                jnp.clip(iv, -_MAX_SIZE, _MAX_SIZE) + _MAX_SIZE
            )

    pltpu.async_copy(emb_hbm.at[idx2.at[0]], rows_v.at[pl.ds(0, half)], gsem).wait()
    pltpu.async_copy(emb_hbm.at[idx2.at[1]], rows_v.at[pl.ds(half, half)], gsem).wait()

    unroll = 8

    def h_step(j, _):
        par = lax.rem(j, 2)
        h_idx = w8 * wb + j
        p0 = h_idx * W
        dst = out_hbm.at[b, pl.ds(p0, W), :]

        @pl.when(j >= 2)
        def _():
            pltpu.make_async_copy(bufs.at[par], dst, csems.at[par]).wait()

        hv = [rows_v[h_idx, pl.ds(t * _L, _L)] for t in range(cvecs)]

        def w_step(i, _):
            for u in range(unroll):
                wc = i * unroll + u
                for t in range(cvecs):
                    bufs[par, wc, pl.ds(t * _L, _L)] = (
                        rows_v[wc, pl.ds(t * _L, _L)] + hv[t]
                    )
            return 0

        lax.fori_loop(0, W // unroll, w_step, 0)
        pltpu.make_async_copy(bufs.at[par], dst, csems.at[par]).start()
        return 0

    lax.fori_loop(0, wb, h_step, 0)

    for t in range(2):
        j = wb - 2 + t
        dst = out_hbm.at[b, pl.ds((w8 * wb + j) * W, W), :]
        pltpu.make_async_copy(bufs.at[lax.rem(j, 2)], dst, csems.at[lax.rem(j, 2)]).wait()


def kernel(feature_map, embedding):
    B, C, H, W = feature_map.shape
    mesh = plsc.VectorSubcoreMesh(core_axis_name="c", subcore_axis_name="s")
    sc_call = pl.kernel(
        functools.partial(_sc_body, B=B, H=H, W=W, C=C),
        out_type=jax.ShapeDtypeStruct((B, H * W, C), jnp.float32),
        mesh=mesh,
        scratch_types=[
            pltpu.VMEM((2, W // 2), jnp.int32),
            pltpu.VMEM((W, 128), jnp.float32),
            pltpu.VMEM((2, W, C), jnp.float32),
            pltpu.SemaphoreType.DMA,
            pltpu.SemaphoreType.DMA((2,)),
        ],
    )
    emb_padded = jnp.pad(embedding, ((0, 0), (0, 128 - C)))
    return sc_call(emb_padded)


# speedup vs baseline: 6.4844x; 6.4844x over previous
import functools

import jax
import jax.numpy as jnp
from jax import lax
from jax.experimental import pallas as pl
from jax.experimental.pallas import tpu as pltpu

_MAX_SIZE = 32


def _clipped_onehot(n_rows, n_idx, base, center):
    row = lax.broadcasted_iota(jnp.int32, (n_rows, n_idx), 0)
    col = lax.broadcasted_iota(jnp.int32, (n_rows, n_idx), 1)
    idx = jnp.clip(base + row - center, -_MAX_SIZE, _MAX_SIZE) + _MAX_SIZE
    return (idx == col).astype(jnp.float32)


def _pos_kernel(emb_ref, out_ref, *, th, h, w, c):
    i = pl.program_id(0)
    n = emb_ref.shape[0]
    emb = emb_ref[...]
    oh_w = _clipped_onehot(w, n, 0, w // 2)
    rows_w = jnp.dot(oh_w, emb, preferred_element_type=jnp.float32)
    oh_h = _clipped_onehot(th, n, i * th, h // 2)
    rows_h = jnp.dot(oh_h, emb, preferred_element_type=jnp.float32)
    for j in range(th):
        out_ref[0, pl.ds(j * w, w), :] = rows_h[j : j + 1, :] + rows_w


def kernel(feature_map, embedding):
    B, C, H, W = feature_map.shape
    TH = 28
    pos = pl.pallas_call(
        functools.partial(_pos_kernel, th=TH, h=H, w=W, c=C),
        grid=(H // TH,),
        in_specs=[
            pl.BlockSpec((embedding.shape[0], C), lambda i: (0, 0)),
        ],
        out_specs=pl.BlockSpec((1, TH * W, C), lambda i: (0, i, 0)),
        out_shape=jax.ShapeDtypeStruct((1, H * W, C), jnp.float32),
        compiler_params=pltpu.CompilerParams(
            dimension_semantics=("parallel",),
        ),
    )(embedding)
    return jnp.broadcast_to(pos, (B, H * W, C))
